# baseline (speedup 1.0000x reference)
import jax
import jax.numpy as jnp
from jax.experimental import pallas as pl
from jax.experimental.pallas import tpu as pltpu

BATCH = 128
MEM = 4096
VAL = 64
B_CH = 1
NCH = BATCH // B_CH
NBUF = 16


def _update_kernel(mem_hbm, w_ref, vt_ref, out_hbm, in_buf, out_buf, in_sems, out_sems):
    def in_copy(c, slot):
        return pltpu.make_async_copy(
            mem_hbm.at[pl.ds(c * B_CH, B_CH)],
            in_buf.at[slot],
            in_sems.at[slot],
        )

    def out_copy(c, slot):
        return pltpu.make_async_copy(
            out_buf.at[slot],
            out_hbm.at[pl.ds(c * B_CH, B_CH)],
            out_sems.at[slot],
        )

    for c in range(NBUF):
        in_copy(c, c).start()

    for c in range(NCH):
        slot = c % NBUF
        in_copy(c, slot).wait()
        if c >= NBUF:
            out_copy(c - NBUF, slot).wait()
        wb = w_ref[pl.ds(c * B_CH, B_CH), :][:, None, :]
        vb = vt_ref[:, pl.ds(c * B_CH, B_CH)].T[:, :, None]
        out_buf[slot] = in_buf[slot] + wb * vb
        out_copy(c, slot).start()
        if c + NBUF < NCH:
            in_copy(c + NBUF, slot).start()

    for c in range(NCH - NBUF, NCH):
        out_copy(c, c % NBUF).wait()


def kernel(memory, w, v):
    mem_t = memory.transpose(0, 2, 1)
    vt = v.T
    out_t = pl.pallas_call(
        _update_kernel,
        in_specs=[
            pl.BlockSpec(memory_space=pltpu.MemorySpace.HBM),
            pl.BlockSpec(memory_space=pltpu.MemorySpace.VMEM),
            pl.BlockSpec(memory_space=pltpu.MemorySpace.VMEM),
        ],
        out_specs=pl.BlockSpec(memory_space=pltpu.MemorySpace.HBM),
        out_shape=jax.ShapeDtypeStruct((BATCH, VAL, MEM), memory.dtype),
        scratch_shapes=[
            pltpu.VMEM((NBUF, B_CH, VAL, MEM), jnp.float32),
            pltpu.VMEM((NBUF, B_CH, VAL, MEM), jnp.float32),
            pltpu.SemaphoreType.DMA((NBUF,)),
            pltpu.SemaphoreType.DMA((NBUF,)),
        ],
    )(mem_t, w, vt)
    return out_t.transpose(0, 2, 1)


# speedup vs baseline: 1.0029x; 1.0029x over previous
import jax
import jax.numpy as jnp
from jax.experimental import pallas as pl
from jax.experimental.pallas import tpu as pltpu

BATCH = 128
MEM = 4096
VAL = 64
B_CH = 4
NCH = BATCH // B_CH
NBUF = 6


def _update_kernel(mem_hbm, w_ref, vt_ref, out_hbm, in_buf, out_buf, in_sems, out_sems):
    def in_copy(c, slot):
        return pltpu.make_async_copy(
            mem_hbm.at[pl.ds(c * B_CH, B_CH)],
            in_buf.at[slot],
            in_sems.at[slot],
        )

    def out_copy(c, slot):
        return pltpu.make_async_copy(
            out_buf.at[slot],
            out_hbm.at[pl.ds(c * B_CH, B_CH)],
            out_sems.at[slot],
        )

    for c in range(NBUF):
        in_copy(c, c).start()

    for c in range(NCH):
        slot = c % NBUF
        in_copy(c, slot).wait()
        if c >= NBUF:
            out_copy(c - NBUF, slot).wait()
        wb = w_ref[pl.ds(c * B_CH, B_CH), :][:, None, :]
        vb = vt_ref[:, pl.ds(c * B_CH, B_CH)].T[:, :, None]
        out_buf[slot] = in_buf[slot] + wb * vb
        out_copy(c, slot).start()
        if c + NBUF < NCH:
            in_copy(c + NBUF, slot).start()

    for c in range(NCH - NBUF, NCH):
        out_copy(c, c % NBUF).wait()


def kernel(memory, w, v):
    mem_t = memory.transpose(0, 2, 1)
    vt = v.T
    out_t = pl.pallas_call(
        _update_kernel,
        in_specs=[
            pl.BlockSpec(memory_space=pltpu.MemorySpace.HBM),
            pl.BlockSpec(memory_space=pltpu.MemorySpace.VMEM),
            pl.BlockSpec(memory_space=pltpu.MemorySpace.VMEM),
        ],
        out_specs=pl.BlockSpec(memory_space=pltpu.MemorySpace.HBM),
        out_shape=jax.ShapeDtypeStruct((BATCH, VAL, MEM), memory.dtype),
        scratch_shapes=[
            pltpu.VMEM((NBUF, B_CH, VAL, MEM), jnp.float32),
            pltpu.VMEM((NBUF, B_CH, VAL, MEM), jnp.float32),
            pltpu.SemaphoreType.DMA((NBUF,)),
            pltpu.SemaphoreType.DMA((NBUF,)),
        ],
    )(mem_t, w, vt)
    return out_t.transpose(0, 2, 1)
